# baseline (speedup 1.0000x reference)
import functools

import jax
import jax.numpy as jnp
from jax import lax
from jax.experimental import pallas as pl
from jax.experimental.pallas import tpu as pltpu
from jax.experimental.pallas import tpu_sc as plsc

_NUM_CAMERAS = 1000
_BATCH = 16384
_D = 16
_TBL = 1024
_RPT = 64
_NC = 2
_NS = 16
_NW = _NC * _NS
_BPW = _BATCH // _NW
_CHUNK = 128
_NCH = _BPW // _CHUNK

_SC_MESH = plsc.VectorSubcoreMesh(core_axis_name="c", subcore_axis_name="s")

_COS = (-0.5, 1 / 24, -1 / 720, 1 / 40320, -1 / 3628800)
_SBT = (-1 / 6, 1 / 120, -1 / 5040, 1 / 362880, -1 / 39916800)
_OMC = (-1 / 24, 1 / 720, -1 / 40320, 1 / 3628800, -1 / 479001600)


def _poly(u, c0, coeffs):
    acc = coeffs[-1]
    for c in reversed(coeffs[:-1]):
        acc = c + u * acc
    return c0 + u * acc


@functools.partial(
    pl.kernel,
    mesh=_SC_MESH,
    out_type=(
        jax.ShapeDtypeStruct((_BATCH * 12,), jnp.float32),
        jax.ShapeDtypeStruct((_TBL, _D), jnp.float32),
    ),
    scratch_types=[
        pltpu.VMEM((_RPT * 6,), jnp.float32),
        pltpu.VMEM((_RPT, _D), jnp.float32),
        pltpu.VMEM((_BPW,), jnp.int32),
        pltpu.VMEM((_BPW, _D), jnp.float32),
        pltpu.VMEM((_BPW * 12,), jnp.float32),
        pltpu.SemaphoreType.DMA,
        pltpu.SemaphoreType.DMA,
        pltpu.SemaphoreType.DMA,
    ],
    compiler_params=pltpu.CompilerParams(
        use_tc_tiling_on_sc=False, needs_layout_passes=False),
)
def _fused_sc(pose_hbm, idx_hbm, out_hbm, tbl_hbm,
              pose_v, table_v, idx_v, rows_v, comp_v, sem, wsem, isem):
    sid = lax.axis_index("s")
    cid = lax.axis_index("c")
    wid = sid * _NC + cid
    base = wid * _BPW
    idx_cp = pltpu.async_copy(idx_hbm.at[pl.ds(base, _BPW)], idx_v, isem)

    @pl.when(sid < _NS - 1)
    def _():
        pltpu.sync_copy(pose_hbm.at[pl.ds(sid * (_RPT * 6), _RPT * 6)], pose_v)

    @pl.when(sid == _NS - 1)
    def _():
        pltpu.sync_copy(pose_hbm.at[pl.ds((_NS - 1) * _RPT * 6, 240)],
                        pose_v.at[pl.ds(0, 240)])

    lanes = lax.iota(jnp.int32, 16)
    for v in range(_RPT // 16):
        b = 96 * v + 6 * lanes
        tx = plsc.load_gather(pose_v, [b])
        ty = plsc.load_gather(pose_v, [b + 1])
        tz = plsc.load_gather(pose_v, [b + 2])
        wx = plsc.load_gather(pose_v, [b + 3])
        wy = plsc.load_gather(pose_v, [b + 4])
        wz = plsc.load_gather(pose_v, [b + 5])
        u = wx * wx + wy * wy + wz * wz
        cosine = _poly(u, 1.0, _COS)
        sbt = _poly(u, 1.0, _SBT)
        omc = _poly(u, 0.5, _OMC)
        swx, swy, swz = sbt * wx, sbt * wy, sbt * wz
        owx, owy = omc * wx, omc * wy
        oxy, oxz, oyz = owx * wy, owx * wz, owy * wz
        vals = (
            owx * wx + cosine, oxy - swz, oxz + swy, tx,
            oxy + swz, owy * wy + cosine, oyz - swx, ty,
            oxz - swy, oyz + swx, omc * wz * wz + cosine, tz,
        )
        rows16 = 16 * v + lanes
        for j, val in enumerate(vals):
            plsc.store_scatter(
                table_v, [rows16, jnp.full((16,), j, jnp.int32)], val)

    @pl.when(sid < _NS - 1)
    def _():
        pltpu.sync_copy(table_v, tbl_hbm.at[pl.ds(sid * _RPT, _RPT)])

    @pl.when(sid == _NS - 1)
    def _():
        pltpu.sync_copy(table_v.at[pl.ds(0, 40)],
                        tbl_hbm.at[pl.ds((_NS - 1) * _RPT, 40)])

    plsc.subcore_barrier()

    idx_cp.wait()
    copies = [
        pltpu.async_copy(
            tbl_hbm.at[idx_v.at[pl.ds(j * _CHUNK, _CHUNK)]],
            rows_v.at[pl.ds(j * _CHUNK, _CHUNK)],
            sem,
        )
        for j in range(_NCH)
    ]
    pat_rows, pat_lanes = [], []
    for p in range(3):
        pos = 16 * p + lanes
        r = pos // 12
        pat_rows.append(r)
        pat_lanes.append(pos - 12 * r)
    writes = []
    for j in range(_NCH):
        copies[j].wait()
        for k in range(_CHUNK * 12 // 16):
            row_idx = pat_rows[k % 3] + (j * _CHUNK + 4 * (k // 3))
            val = plsc.load_gather(rows_v, [row_idx, pat_lanes[k % 3]])
            comp_v[pl.ds((j * _CHUNK * 12 // 16 + k) * 16, 16)] = val
        writes.append(
            pltpu.async_copy(
                comp_v.at[pl.ds(j * _CHUNK * 12, _CHUNK * 12)],
                out_hbm.at[pl.ds((base + j * _CHUNK) * 12, _CHUNK * 12)],
                wsem,
            )
        )
    for w in writes:
        w.wait()


def kernel(indices, pose_adjustment):
    pose_flat = pose_adjustment.astype(jnp.float32).reshape(_NUM_CAMERAS * 6)
    idx32 = indices.astype(jnp.int32)
    full, _ = _fused_sc(pose_flat, idx32)
    return full.reshape(_BATCH, 3, 4)


# speedup vs baseline: 2.7537x; 2.7537x over previous
import functools

import jax
import jax.numpy as jnp
from jax import lax
from jax.experimental import pallas as pl
from jax.experimental.pallas import tpu as pltpu
from jax.experimental.pallas import tpu_sc as plsc

_NUM_CAMERAS = 1000
_BATCH = 16384
_D = 16
_TBL = 1024
_RPT = 64
_NC = 2
_NS = 16
_NW = _NC * _NS
_BPW = _BATCH // _NW
_CHUNK = 128
_NCH = _BPW // _CHUNK

_SC_MESH = plsc.VectorSubcoreMesh(core_axis_name="c", subcore_axis_name="s")

_COS = (-0.5, 1 / 24, -1 / 720, 1 / 40320, -1 / 3628800)
_SBT = (-1 / 6, 1 / 120, -1 / 5040, 1 / 362880, -1 / 39916800)
_OMC = (-1 / 24, 1 / 720, -1 / 40320, 1 / 3628800, -1 / 479001600)


def _poly(u, c0, coeffs):
    acc = coeffs[-1]
    for c in reversed(coeffs[:-1]):
        acc = c + u * acc
    return c0 + u * acc


@functools.partial(
    pl.kernel,
    mesh=_SC_MESH,
    out_type=(
        jax.ShapeDtypeStruct((_BATCH, _D), jnp.float32),
        jax.ShapeDtypeStruct((_TBL, _D), jnp.float32),
    ),
    scratch_types=[
        pltpu.VMEM((_RPT * 6,), jnp.float32),
        pltpu.VMEM((_RPT, _D), jnp.float32),
        pltpu.VMEM((_BPW,), jnp.int32),
        pltpu.VMEM((_BPW, _D), jnp.float32),
        pltpu.SemaphoreType.DMA,
        pltpu.SemaphoreType.DMA,
        pltpu.SemaphoreType.DMA,
    ],
    compiler_params=pltpu.CompilerParams(
        use_tc_tiling_on_sc=False, needs_layout_passes=False),
)
def _fused_sc(pose_hbm, idx_hbm, out_hbm, tbl_hbm,
              pose_v, table_v, idx_v, rows_v, sem, wsem, isem):
    sid = lax.axis_index("s")
    cid = lax.axis_index("c")
    wid = sid * _NC + cid
    base = wid * _BPW
    idx_cp = pltpu.async_copy(idx_hbm.at[pl.ds(base, _BPW)], idx_v, isem)

    @pl.when(sid < _NS - 1)
    def _():
        pltpu.sync_copy(pose_hbm.at[pl.ds(sid * (_RPT * 6), _RPT * 6)], pose_v)

    @pl.when(sid == _NS - 1)
    def _():
        pltpu.sync_copy(pose_hbm.at[pl.ds((_NS - 1) * _RPT * 6, 240)],
                        pose_v.at[pl.ds(0, 240)])

    lanes = lax.iota(jnp.int32, 16)
    for v in range(_RPT // 16):
        b = 96 * v + 6 * lanes
        tx = plsc.load_gather(pose_v, [b])
        ty = plsc.load_gather(pose_v, [b + 1])
        tz = plsc.load_gather(pose_v, [b + 2])
        wx = plsc.load_gather(pose_v, [b + 3])
        wy = plsc.load_gather(pose_v, [b + 4])
        wz = plsc.load_gather(pose_v, [b + 5])
        u = wx * wx + wy * wy + wz * wz
        cosine = _poly(u, 1.0, _COS)
        sbt = _poly(u, 1.0, _SBT)
        omc = _poly(u, 0.5, _OMC)
        swx, swy, swz = sbt * wx, sbt * wy, sbt * wz
        owx, owy = omc * wx, omc * wy
        oxy, oxz, oyz = owx * wy, owx * wz, owy * wz
        vals = (
            owx * wx + cosine, oxy - swz, oxz + swy, tx,
            oxy + swz, owy * wy + cosine, oyz - swx, ty,
            oxz - swy, oyz + swx, omc * wz * wz + cosine, tz,
        )
        rows16 = 16 * v + lanes
        for j, val in enumerate(vals):
            plsc.store_scatter(
                table_v, [rows16, jnp.full((16,), j, jnp.int32)], val)

    @pl.when(sid < _NS - 1)
    def _():
        pltpu.sync_copy(table_v, tbl_hbm.at[pl.ds(sid * _RPT, _RPT)])

    @pl.when(sid == _NS - 1)
    def _():
        pltpu.sync_copy(table_v.at[pl.ds(0, 40)],
                        tbl_hbm.at[pl.ds((_NS - 1) * _RPT, 40)])

    plsc.subcore_barrier()

    idx_cp.wait()
    copies = [
        pltpu.async_copy(
            tbl_hbm.at[idx_v.at[pl.ds(j * _CHUNK, _CHUNK)]],
            rows_v.at[pl.ds(j * _CHUNK, _CHUNK)],
            sem,
        )
        for j in range(_NCH)
    ]
    writes = []
    for j in range(_NCH):
        copies[j].wait()
        writes.append(
            pltpu.async_copy(
                rows_v.at[pl.ds(j * _CHUNK, _CHUNK)],
                out_hbm.at[pl.ds(base + j * _CHUNK, _CHUNK)],
                wsem,
            )
        )
    for w in writes:
        w.wait()


def kernel(indices, pose_adjustment):
    pose_flat = pose_adjustment.astype(jnp.float32).reshape(_NUM_CAMERAS * 6)
    idx32 = indices.astype(jnp.int32)
    full, _ = _fused_sc(pose_flat, idx32)
    return full.reshape(_BATCH, 4, 4)[:, :3, :]
